# baseline (speedup 1.0000x reference)
import functools
import math

import jax
import jax.numpy as jnp
from jax import lax
from jax.experimental import pallas as pl
from jax.experimental.pallas import tpu as pltpu
from jax.experimental.pallas import tpu_sc as plsc

N = 2048
E = 32768
F_IN = 128
H = 16

NC = 2
NS = 16
NW = NC * NS
EPW = E // NW
CHUNK = 128
NCHUNK = EPW // CHUNK
RPT = N // NS

RB = 256
NRB = N // RB

_F32 = jnp.float32

@functools.lru_cache(maxsize=None)
def _sc_mesh():
    return plsc.VectorSubcoreMesh(
        core_axis_name="c", subcore_axis_name="s",
        num_cores=NC, num_subcores=NS)


def _fill_const(ref, value, nrows):
    def body(i, _):
        ref[i] = jnp.full((H,), value, _F32)
        return 0
    lax.fori_loop(0, nrows, body, 0)


def _deg_body(ei_hbm, ones_hbm, zeros_hbm, out_hbm,
              idx_v, ones_v, zeros_v, acc_sh, sem):
    c = lax.axis_index("c")
    s = lax.axis_index("s")
    wid = c * NS + s
    base = wid * EPW
    pltpu.sync_copy(ones_hbm, ones_v)
    pltpu.sync_copy(zeros_hbm, zeros_v)
    for j in range(NCHUNK):
        pltpu.sync_copy(ei_hbm.at[1, pl.ds(base + j * CHUNK, CHUNK)],
                        idx_v.at[j])
    pltpu.sync_copy(zeros_v, acc_sh.at[pl.ds(s * RPT, RPT)])
    plsc.subcore_barrier()
    descs = [pltpu.async_copy(ones_v, acc_sh.at[idx_v.at[j]], sem, add=True)
             for j in range(NCHUNK)]
    for d in descs:
        d.wait()
    plsc.subcore_barrier()
    pltpu.sync_copy(acc_sh.at[pl.ds(s * RPT, RPT)],
                    out_hbm.at[c].at[pl.ds(s * RPT, RPT)])


@functools.lru_cache(maxsize=None)
def _deg_call():
    return pl.kernel(
        _deg_body,
        out_type=jax.ShapeDtypeStruct((NC, N, H), _F32),
        mesh=_sc_mesh(),
        scratch_types=[
            pltpu.VMEM((NCHUNK, CHUNK), jnp.int32),
            pltpu.VMEM((CHUNK, H), _F32),
            pltpu.VMEM((RPT, H), _F32),
            pltpu.VMEM_SHARED((N, H), _F32),
            pltpu.SemaphoreType.DMA,
        ],
        compiler_params=pltpu.CompilerParams(use_tc_tiling_on_sc=False),
    )


def _seg_body(ei_hbm, xws_hbm, out_hbm,
              idxs_v, idxd_v, rows_v, zeros_v, acc_sh, sem0, sem1):
    c = lax.axis_index("c")
    s = lax.axis_index("s")
    wid = c * NS + s
    base = wid * EPW
    _fill_const(zeros_v, 0.0, RPT)
    for j in range(NCHUNK):
        pltpu.sync_copy(ei_hbm.at[0, pl.ds(base + j * CHUNK, CHUNK)],
                        idxs_v.at[j])
        pltpu.sync_copy(ei_hbm.at[1, pl.ds(base + j * CHUNK, CHUNK)],
                        idxd_v.at[j])
    pltpu.sync_copy(zeros_v, acc_sh.at[pl.ds(s * RPT, RPT)])
    plsc.subcore_barrier()
    sems = (sem0, sem1)
    desc = pltpu.async_copy(xws_hbm.at[idxs_v.at[0]], rows_v.at[0], sems[0])
    for j in range(NCHUNK):
        desc.wait()
        if j + 1 < NCHUNK:
            desc = pltpu.async_copy(xws_hbm.at[idxs_v.at[j + 1]],
                                    rows_v.at[(j + 1) % 2],
                                    sems[(j + 1) % 2])
        pltpu.sync_copy(rows_v.at[j % 2], acc_sh.at[idxd_v.at[j]], add=True)
    plsc.subcore_barrier()
    pltpu.sync_copy(acc_sh.at[pl.ds(s * RPT, RPT)],
                    out_hbm.at[c].at[pl.ds(s * RPT, RPT)])


@functools.lru_cache(maxsize=None)
def _seg_call():
    return pl.kernel(
        _seg_body,
        out_type=jax.ShapeDtypeStruct((NC, N, H), _F32),
        mesh=_sc_mesh(),
        scratch_types=[
            pltpu.VMEM((NCHUNK, CHUNK), jnp.int32),
            pltpu.VMEM((NCHUNK, CHUNK), jnp.int32),
            pltpu.VMEM((2, CHUNK, H), _F32),
            pltpu.VMEM((RPT, H), _F32),
            pltpu.VMEM_SHARED((N, H), _F32),
            pltpu.SemaphoreType.DMA,
            pltpu.SemaphoreType.DMA,
        ],
        compiler_params=pltpu.CompilerParams(use_tc_tiling_on_sc=False),
    )


def _prep_body(x_ref, w1_ref, deg2_ref, xws_ref, dinv_ref, p2_ref):
    deg = deg2_ref[0, :, 0:1] + deg2_ref[1, :, 0:1] + 1.0
    dinv = lax.rsqrt(deg)
    xw = jnp.dot(x_ref[...], w1_ref[...], preferred_element_type=_F32)
    xws = dinv * xw
    xws_ref[...] = xws
    dinv_ref[...] = jnp.broadcast_to(dinv, (N, H))
    p2_ref[...] = dinv * xws


def _prep_call(x, W1, deg2):
    return pl.pallas_call(
        _prep_body,
        out_shape=(
            jax.ShapeDtypeStruct((N, H), _F32),
            jax.ShapeDtypeStruct((N, H), _F32),
            jax.ShapeDtypeStruct((N, H), _F32),
        ),
    )(x, W1, deg2)


def _ab_body(s2_ref, dinv_ref, p2_ref, b1_ref, wt_ref, wb_ref, bc1_ref,
             a_ref, b_ref, at_ref, bt_ref):
    S = s2_ref[0] + s2_ref[1]
    h = jnp.maximum(dinv_ref[...] * S + p2_ref[...] + b1_ref[...], 0.0)
    A = jnp.dot(h, wt_ref[...], preferred_element_type=_F32) + bc1_ref[...]
    B = jnp.dot(h, wb_ref[...], preferred_element_type=_F32)
    a_ref[...] = A
    b_ref[...] = B
    at_ref[...] = A.T
    bt_ref[...] = B.T


def _ab_call(S2, dinv_b, p2, b1, Wt, Wb, bc1):
    return pl.pallas_call(
        _ab_body,
        out_shape=(
            jax.ShapeDtypeStruct((N, H), _F32),
            jax.ShapeDtypeStruct((N, H), _F32),
            jax.ShapeDtypeStruct((H, N), _F32),
            jax.ShapeDtypeStruct((H, N), _F32),
        ),
    )(S2, dinv_b, p2, b1, Wt, Wb, bc1)


def _adj_body(a_ref, b_ref, at_ref, bt_ref, w2_ref, tadj_ref, out_ref):
    bi = pl.program_id(0)
    t = tadj_ref[0]
    a = a_ref[...]
    b = b_ref[...]
    a_bc = [jnp.broadcast_to(a[:, k:k + 1], (RB, RB)) for k in range(H)]
    b_bc = [jnp.broadcast_to(b[:, k:k + 1], (RB, RB)) for k in range(H)]

    def half_v(rows_bc, ncols):
        acc = jnp.zeros((RB, RB), _F32)
        r = jnp.zeros((1, RB), _F32)
        for k in range(H):
            w = w2_ref[k]
            acc = acc + jnp.maximum(rows_bc[k], ncols[k:k + 1, :]) * w
            r = r - ncols[k:k + 1, :] * w
        return acc + r

    for j in range(NRB):
        sl = slice(j * RB, (j + 1) * RB)
        natj = -at_ref[:, sl]
        nbtj = -bt_ref[:, sl]

        @pl.when(bi < j)
        def _():
            v = half_v(a_bc, nbtj)
            out_ref[:, sl] = jnp.where(v > t, 1.0, 0.0)

        @pl.when(bi > j)
        def _():
            v = half_v(b_bc, natj)
            out_ref[:, sl] = jnp.where(v > t, 1.0, 0.0)

        @pl.when(bi == j)
        def _():
            v1 = half_v(a_bc, nbtj)
            v2 = half_v(b_bc, natj)
            row = lax.broadcasted_iota(jnp.int32, (RB, RB), 0)
            col = lax.broadcasted_iota(jnp.int32, (RB, RB), 1)
            u1 = jnp.where(v1 > t, 1.0, 0.0)
            u2 = jnp.where(v2 > t, 1.0, 0.0)
            out_ref[:, sl] = jnp.where(col > row, u1,
                                       jnp.where(col < row, u2, 1.0))


def _adj_call(A, B, At, Bt, w2, tadj):
    return pl.pallas_call(
        _adj_body,
        grid=(NRB,),
        in_specs=[
            pl.BlockSpec((RB, H), lambda i: (i, 0)),
            pl.BlockSpec((RB, H), lambda i: (i, 0)),
            pl.BlockSpec((H, N), lambda i: (0, 0)),
            pl.BlockSpec((H, N), lambda i: (0, 0)),
            pl.BlockSpec(memory_space=pltpu.SMEM),
            pl.BlockSpec(memory_space=pltpu.SMEM),
        ],
        out_specs=pl.BlockSpec((RB, N), lambda i: (i, 0)),
        out_shape=jax.ShapeDtypeStruct((N, N), _F32),
    )(A, B, At, Bt, w2, tadj)


def kernel(x, edge_index, W1, b1, Wc1, bc1, Wc2, bc2):
    ones128 = jnp.ones((CHUNK, H), _F32)
    zeros128 = jnp.zeros((RPT, H), _F32)
    deg2 = _deg_call()(edge_index, ones128, zeros128)
    xws, dinv_b, p2 = _prep_call(x, W1, deg2)
    S2 = _seg_call()(edge_index, xws)
    A, B, At, Bt = _ab_call(S2, dinv_b, p2, b1.reshape(1, H),
                            Wc1[:H], Wc1[H:], bc1.reshape(1, H))
    w2 = Wc2[:, 0]
    tadj = jnp.float32(math.log(0.3 / 0.7)) - bc2
    return _adj_call(A, B, At, Bt, w2, tadj)


# speedup vs baseline: 1.0955x; 1.0955x over previous
import functools
import math

import jax
import jax.numpy as jnp
from jax import lax
from jax.experimental import pallas as pl
from jax.experimental.pallas import tpu as pltpu
from jax.experimental.pallas import tpu_sc as plsc

N = 2048
E = 32768
F_IN = 128
H = 16

NC = 2
NS = 16
NW = NC * NS
EPW = E // NW
CHUNK = 128
NCHUNK = EPW // CHUNK
RPT = N // NS

RB = 256
NRB = N // RB

_F32 = jnp.float32

@functools.lru_cache(maxsize=None)
def _sc_mesh():
    return plsc.VectorSubcoreMesh(
        core_axis_name="c", subcore_axis_name="s",
        num_cores=NC, num_subcores=NS)


def _fill_const(ref, value, nrows):
    def body(i, _):
        ref[i] = jnp.full((H,), value, _F32)
        return 0
    lax.fori_loop(0, nrows, body, 0)


def _deg_body(dst_hbm, ones_hbm, zeros_hbm, out_hbm,
              idx_v, ones_v, zeros_v, acc_sh, sem):
    c = lax.axis_index("c")
    s = lax.axis_index("s")
    wid = c * NS + s
    pltpu.sync_copy(ones_hbm, ones_v)
    pltpu.sync_copy(zeros_hbm, zeros_v)
    pltpu.sync_copy(dst_hbm.at[wid], idx_v)
    pltpu.sync_copy(zeros_v, acc_sh.at[pl.ds(s * RPT, RPT)])
    plsc.subcore_barrier()
    descs = [pltpu.async_copy(ones_v, acc_sh.at[idx_v.at[j]], sem, add=True)
             for j in range(NCHUNK)]
    for d in descs:
        d.wait()
    plsc.subcore_barrier()
    pltpu.sync_copy(acc_sh.at[pl.ds(s * RPT, RPT)],
                    out_hbm.at[c].at[pl.ds(s * RPT, RPT)])


@functools.lru_cache(maxsize=None)
def _deg_call():
    return pl.kernel(
        _deg_body,
        out_type=jax.ShapeDtypeStruct((NC, N, H), _F32),
        mesh=_sc_mesh(),
        scratch_types=[
            pltpu.VMEM((NCHUNK, CHUNK), jnp.int32),
            pltpu.VMEM((CHUNK, H), _F32),
            pltpu.VMEM((RPT, H), _F32),
            pltpu.VMEM_SHARED((N, H), _F32),
            pltpu.SemaphoreType.DMA,
        ],
        compiler_params=pltpu.CompilerParams(use_tc_tiling_on_sc=False),
    )


def _seg_body(src_hbm, dst_hbm, xws_hbm, out_hbm,
              idxs_v, idxd_v, rows_v, zeros_v, acc_sh, sem0, sem1):
    c = lax.axis_index("c")
    s = lax.axis_index("s")
    wid = c * NS + s
    _fill_const(zeros_v, 0.0, RPT)
    pltpu.sync_copy(src_hbm.at[wid], idxs_v)
    pltpu.sync_copy(dst_hbm.at[wid], idxd_v)
    pltpu.sync_copy(zeros_v, acc_sh.at[pl.ds(s * RPT, RPT)])
    plsc.subcore_barrier()
    sems = (sem0, sem1)
    desc = pltpu.async_copy(xws_hbm.at[idxs_v.at[0]], rows_v.at[0], sems[0])
    for j in range(NCHUNK):
        desc.wait()
        if j + 1 < NCHUNK:
            desc = pltpu.async_copy(xws_hbm.at[idxs_v.at[j + 1]],
                                    rows_v.at[(j + 1) % 2],
                                    sems[(j + 1) % 2])
        pltpu.sync_copy(rows_v.at[j % 2], acc_sh.at[idxd_v.at[j]], add=True)
    plsc.subcore_barrier()
    pltpu.sync_copy(acc_sh.at[pl.ds(s * RPT, RPT)],
                    out_hbm.at[c].at[pl.ds(s * RPT, RPT)])


@functools.lru_cache(maxsize=None)
def _seg_call():
    return pl.kernel(
        _seg_body,
        out_type=jax.ShapeDtypeStruct((NC, N, H), _F32),
        mesh=_sc_mesh(),
        scratch_types=[
            pltpu.VMEM((NCHUNK, CHUNK), jnp.int32),
            pltpu.VMEM((NCHUNK, CHUNK), jnp.int32),
            pltpu.VMEM((2, CHUNK, H), _F32),
            pltpu.VMEM((RPT, H), _F32),
            pltpu.VMEM_SHARED((N, H), _F32),
            pltpu.SemaphoreType.DMA,
            pltpu.SemaphoreType.DMA,
        ],
        compiler_params=pltpu.CompilerParams(use_tc_tiling_on_sc=False),
    )


def _prep_body(x_ref, w1_ref, deg2_ref, xws_ref, dinv_ref, p2_ref):
    deg = deg2_ref[0, :, 0:1] + deg2_ref[1, :, 0:1] + 1.0
    dinv = lax.rsqrt(deg)
    xw = jnp.dot(x_ref[...], w1_ref[...], preferred_element_type=_F32)
    xws = dinv * xw
    xws_ref[...] = xws
    dinv_ref[...] = jnp.broadcast_to(dinv, (N, H))
    p2_ref[...] = dinv * xws


def _prep_call(x, W1, deg2):
    return pl.pallas_call(
        _prep_body,
        out_shape=(
            jax.ShapeDtypeStruct((N, H), _F32),
            jax.ShapeDtypeStruct((N, H), _F32),
            jax.ShapeDtypeStruct((N, H), _F32),
        ),
    )(x, W1, deg2)


def _ab_body(s2_ref, dinv_ref, p2_ref, b1_ref, wt_ref, wb_ref, bc1_ref,
             a_ref, b_ref, at_ref, bt_ref):
    S = s2_ref[0] + s2_ref[1]
    h = jnp.maximum(dinv_ref[...] * S + p2_ref[...] + b1_ref[...], 0.0)
    A = jnp.dot(h, wt_ref[...], preferred_element_type=_F32) + bc1_ref[...]
    B = jnp.dot(h, wb_ref[...], preferred_element_type=_F32)
    a_ref[...] = A
    b_ref[...] = B
    at_ref[...] = A.T
    bt_ref[...] = B.T


def _ab_call(S2, dinv_b, p2, b1, Wt, Wb, bc1):
    return pl.pallas_call(
        _ab_body,
        out_shape=(
            jax.ShapeDtypeStruct((N, H), _F32),
            jax.ShapeDtypeStruct((N, H), _F32),
            jax.ShapeDtypeStruct((H, N), _F32),
            jax.ShapeDtypeStruct((H, N), _F32),
        ),
    )(S2, dinv_b, p2, b1, Wt, Wb, bc1)


def _adj_body(a_ref, b_ref, at_ref, bt_ref, w2_ref, tadj_ref, out_ref):
    bi = pl.program_id(0)
    t = tadj_ref[0]
    a = a_ref[...]
    b = b_ref[...]
    a_bc = [jnp.broadcast_to(a[:, k:k + 1], (RB, RB)) for k in range(H)]
    b_bc = [jnp.broadcast_to(b[:, k:k + 1], (RB, RB)) for k in range(H)]

    def half_v(rows_bc, ncols):
        acc = jnp.zeros((RB, RB), _F32)
        r = jnp.zeros((1, RB), _F32)
        for k in range(H):
            w = w2_ref[k]
            acc = acc + jnp.maximum(rows_bc[k], ncols[k:k + 1, :]) * w
            r = r - ncols[k:k + 1, :] * w
        return acc + r

    for j in range(NRB):
        sl = slice(j * RB, (j + 1) * RB)
        natj = -at_ref[:, sl]
        nbtj = -bt_ref[:, sl]

        @pl.when(bi < j)
        def _():
            v = half_v(a_bc, nbtj)
            out_ref[:, sl] = jnp.where(v > t, 1.0, 0.0)

        @pl.when(bi > j)
        def _():
            v = half_v(b_bc, natj)
            out_ref[:, sl] = jnp.where(v > t, 1.0, 0.0)

        @pl.when(bi == j)
        def _():
            v1 = half_v(a_bc, nbtj)
            v2 = half_v(b_bc, natj)
            row = lax.broadcasted_iota(jnp.int32, (RB, RB), 0)
            col = lax.broadcasted_iota(jnp.int32, (RB, RB), 1)
            u1 = jnp.where(v1 > t, 1.0, 0.0)
            u2 = jnp.where(v2 > t, 1.0, 0.0)
            out_ref[:, sl] = jnp.where(col > row, u1,
                                       jnp.where(col < row, u2, 1.0))


def _adj_call(A, B, At, Bt, w2, tadj):
    return pl.pallas_call(
        _adj_body,
        grid=(NRB,),
        in_specs=[
            pl.BlockSpec((RB, H), lambda i: (i, 0)),
            pl.BlockSpec((RB, H), lambda i: (i, 0)),
            pl.BlockSpec((H, N), lambda i: (0, 0)),
            pl.BlockSpec((H, N), lambda i: (0, 0)),
            pl.BlockSpec(memory_space=pltpu.SMEM),
            pl.BlockSpec(memory_space=pltpu.SMEM),
        ],
        out_specs=pl.BlockSpec((RB, N), lambda i: (i, 0)),
        out_shape=jax.ShapeDtypeStruct((N, N), _F32),
    )(A, B, At, Bt, w2, tadj)


def kernel(x, edge_index, W1, b1, Wc1, bc1, Wc2, bc2):
    src = edge_index[0].reshape(NW, NCHUNK, CHUNK)
    dst = edge_index[1].reshape(NW, NCHUNK, CHUNK)
    ones128 = jnp.ones((CHUNK, H), _F32)
    zeros128 = jnp.zeros((RPT, H), _F32)
    deg2 = _deg_call()(dst, ones128, zeros128)
    xws, dinv_b, p2 = _prep_call(x, W1, deg2)
    S2 = _seg_call()(src, dst, xws)
    A, B, At, Bt = _ab_call(S2, dinv_b, p2, b1.reshape(1, H),
                            Wc1[:H], Wc1[H:], bc1.reshape(1, H))
    w2 = Wc2[:, 0]
    tadj = jnp.float32(math.log(0.3 / 0.7)) - bc2
    return _adj_call(A, B, At, Bt, w2, tadj)


# speedup vs baseline: 1.1226x; 1.0247x over previous
import functools
import math

import jax
import jax.numpy as jnp
from jax import lax
from jax.experimental import pallas as pl
from jax.experimental.pallas import tpu as pltpu
from jax.experimental.pallas import tpu_sc as plsc

N = 2048
E = 32768
F_IN = 128
H = 16

NC = 2
NS = 16
NW = NC * NS
EPW = E // NW
CHUNK = 128
NCHUNK = EPW // CHUNK
RPT = N // NS

RB = 256
NRB = N // RB

_F32 = jnp.float32

@functools.lru_cache(maxsize=None)
def _sc_mesh():
    return plsc.VectorSubcoreMesh(
        core_axis_name="c", subcore_axis_name="s",
        num_cores=NC, num_subcores=NS)


def _fill_const(ref, value, nrows):
    def body(i, _):
        ref[i] = jnp.full((H,), value, _F32)
        return 0
    lax.fori_loop(0, nrows, body, 0)


def _deg_body(dst_hbm, ones_hbm, zeros_hbm, out_hbm,
              idx_v, ones_v, zeros_v, acc_sh, sem):
    c = lax.axis_index("c")
    s = lax.axis_index("s")
    wid = c * NS + s
    pltpu.sync_copy(ones_hbm, ones_v)
    pltpu.sync_copy(zeros_hbm, zeros_v)
    pltpu.sync_copy(dst_hbm.at[wid], idx_v)
    pltpu.sync_copy(zeros_v, acc_sh.at[pl.ds(s * RPT, RPT)])
    plsc.subcore_barrier()
    descs = [pltpu.async_copy(ones_v, acc_sh.at[idx_v.at[j]], sem, add=True)
             for j in range(NCHUNK)]
    for d in descs:
        d.wait()
    plsc.subcore_barrier()
    pltpu.sync_copy(acc_sh.at[pl.ds(s * RPT, RPT)],
                    out_hbm.at[c].at[pl.ds(s * RPT, RPT)])


@functools.lru_cache(maxsize=None)
def _deg_call():
    return pl.kernel(
        _deg_body,
        out_type=jax.ShapeDtypeStruct((NC, N, H), _F32),
        mesh=_sc_mesh(),
        scratch_types=[
            pltpu.VMEM((NCHUNK, CHUNK), jnp.int32),
            pltpu.VMEM((CHUNK, H), _F32),
            pltpu.VMEM((RPT, H), _F32),
            pltpu.VMEM_SHARED((N, H), _F32),
            pltpu.SemaphoreType.DMA,
        ],
        compiler_params=pltpu.CompilerParams(use_tc_tiling_on_sc=False),
    )


def _seg_body(src_hbm, dst_hbm, xws_hbm, out_hbm,
              idxs_v, idxd_v, rows_v, zeros_v, acc_sh, sem0, sem1):
    c = lax.axis_index("c")
    s = lax.axis_index("s")
    wid = c * NS + s
    _fill_const(zeros_v, 0.0, RPT)
    pltpu.sync_copy(src_hbm.at[wid], idxs_v)
    pltpu.sync_copy(dst_hbm.at[wid], idxd_v)
    pltpu.sync_copy(zeros_v, acc_sh.at[pl.ds(s * RPT, RPT)])
    plsc.subcore_barrier()
    sems = (sem0, sem1)
    desc = pltpu.async_copy(xws_hbm.at[idxs_v.at[0]], rows_v.at[0], sems[0])
    for j in range(NCHUNK):
        desc.wait()
        if j + 1 < NCHUNK:
            desc = pltpu.async_copy(xws_hbm.at[idxs_v.at[j + 1]],
                                    rows_v.at[(j + 1) % 2],
                                    sems[(j + 1) % 2])
        pltpu.sync_copy(rows_v.at[j % 2], acc_sh.at[idxd_v.at[j]], add=True)
    plsc.subcore_barrier()
    pltpu.sync_copy(acc_sh.at[pl.ds(s * RPT, RPT)],
                    out_hbm.at[c].at[pl.ds(s * RPT, RPT)])


@functools.lru_cache(maxsize=None)
def _seg_call():
    return pl.kernel(
        _seg_body,
        out_type=jax.ShapeDtypeStruct((NC, N, H), _F32),
        mesh=_sc_mesh(),
        scratch_types=[
            pltpu.VMEM((NCHUNK, CHUNK), jnp.int32),
            pltpu.VMEM((NCHUNK, CHUNK), jnp.int32),
            pltpu.VMEM((2, CHUNK, H), _F32),
            pltpu.VMEM((RPT, H), _F32),
            pltpu.VMEM_SHARED((N, H), _F32),
            pltpu.SemaphoreType.DMA,
            pltpu.SemaphoreType.DMA,
        ],
        compiler_params=pltpu.CompilerParams(use_tc_tiling_on_sc=False),
    )


def _prep_body(x_ref, w1_ref, deg2_ref, xws_ref, dinv_ref, p2_ref):
    deg = deg2_ref[0, :, 0:1] + deg2_ref[1, :, 0:1] + 1.0
    dinv = lax.rsqrt(deg)
    xw = jnp.dot(x_ref[...], w1_ref[...], preferred_element_type=_F32)
    xws = dinv * xw
    xws_ref[...] = xws
    dinv_ref[...] = jnp.broadcast_to(dinv, (N, H))
    p2_ref[...] = dinv * xws


def _prep_call(x, W1, deg2):
    return pl.pallas_call(
        _prep_body,
        out_shape=(
            jax.ShapeDtypeStruct((N, H), _F32),
            jax.ShapeDtypeStruct((N, H), _F32),
            jax.ShapeDtypeStruct((N, H), _F32),
        ),
    )(x, W1, deg2)


def _adj_body(s2_ref, dinv_ref, p2_ref, b1_ref, wt_ref, wb_ref, bc1_ref,
              w2_ref, tadj_ref, out_ref, a_s, b_s, at_s, bt_s):
    bi = pl.program_id(0)

    @pl.when(bi == 0)
    def _():
        S = s2_ref[0] + s2_ref[1]
        h = jnp.maximum(dinv_ref[...] * S + p2_ref[...] + b1_ref[...], 0.0)
        A = jnp.dot(h, wt_ref[...], preferred_element_type=_F32) \
            + bc1_ref[...]
        B = jnp.dot(h, wb_ref[...], preferred_element_type=_F32)
        a_s[...] = A
        b_s[...] = B
        at_s[...] = A.T
        bt_s[...] = B.T

    t = tadj_ref[0]
    a = a_s[pl.ds(bi * RB, RB), :]
    b = b_s[pl.ds(bi * RB, RB), :]
    a_bc = [jnp.broadcast_to(a[:, k:k + 1], (RB, RB)) for k in range(H)]
    b_bc = [jnp.broadcast_to(b[:, k:k + 1], (RB, RB)) for k in range(H)]

    def half_v(rows_bc, ncols):
        acc = jnp.zeros((RB, RB), _F32)
        r = jnp.zeros((1, RB), _F32)
        for k in range(H):
            w = w2_ref[k]
            acc = acc + jnp.maximum(rows_bc[k], ncols[k:k + 1, :]) * w
            r = r - ncols[k:k + 1, :] * w
        return acc + r

    for j in range(NRB):
        sl = slice(j * RB, (j + 1) * RB)
        natj = -at_s[:, sl]
        nbtj = -bt_s[:, sl]

        @pl.when(bi < j)
        def _():
            v = half_v(a_bc, nbtj)
            out_ref[:, sl] = jnp.where(v > t, 1.0, 0.0)

        @pl.when(bi > j)
        def _():
            v = half_v(b_bc, natj)
            out_ref[:, sl] = jnp.where(v > t, 1.0, 0.0)

        @pl.when(bi == j)
        def _():
            v1 = half_v(a_bc, nbtj)
            v2 = half_v(b_bc, natj)
            row = lax.broadcasted_iota(jnp.int32, (RB, RB), 0)
            col = lax.broadcasted_iota(jnp.int32, (RB, RB), 1)
            u1 = jnp.where(v1 > t, 1.0, 0.0)
            u2 = jnp.where(v2 > t, 1.0, 0.0)
            out_ref[:, sl] = jnp.where(col > row, u1,
                                       jnp.where(col < row, u2, 1.0))


def _adj_call(S2, dinv_b, p2, b1, Wt, Wb, bc1, w2, tadj):
    return pl.pallas_call(
        _adj_body,
        grid=(NRB,),
        in_specs=[
            pl.BlockSpec((2, N, H), lambda i: (0, 0, 0)),
            pl.BlockSpec((N, H), lambda i: (0, 0)),
            pl.BlockSpec((N, H), lambda i: (0, 0)),
            pl.BlockSpec((1, H), lambda i: (0, 0)),
            pl.BlockSpec((H, H), lambda i: (0, 0)),
            pl.BlockSpec((H, H), lambda i: (0, 0)),
            pl.BlockSpec((1, H), lambda i: (0, 0)),
            pl.BlockSpec(memory_space=pltpu.SMEM),
            pl.BlockSpec(memory_space=pltpu.SMEM),
        ],
        out_specs=pl.BlockSpec((RB, N), lambda i: (i, 0)),
        out_shape=jax.ShapeDtypeStruct((N, N), _F32),
        scratch_shapes=[
            pltpu.VMEM((N, H), _F32),
            pltpu.VMEM((N, H), _F32),
            pltpu.VMEM((H, N), _F32),
            pltpu.VMEM((H, N), _F32),
        ],
    )(S2, dinv_b, p2, b1, Wt, Wb, bc1, w2, tadj)


def kernel(x, edge_index, W1, b1, Wc1, bc1, Wc2, bc2):
    src = edge_index[0].reshape(NW, NCHUNK, CHUNK)
    dst = edge_index[1].reshape(NW, NCHUNK, CHUNK)
    ones128 = jnp.ones((CHUNK, H), _F32)
    zeros128 = jnp.zeros((RPT, H), _F32)
    deg2 = _deg_call()(dst, ones128, zeros128)
    xws, dinv_b, p2 = _prep_call(x, W1, deg2)
    S2 = _seg_call()(src, dst, xws)
    w2 = Wc2[:, 0]
    tadj = jnp.float32(math.log(0.3 / 0.7)) - bc2
    return _adj_call(S2, dinv_b, p2, b1.reshape(1, H),
                     Wc1[:H], Wc1[H:], bc1.reshape(1, H), w2, tadj)
